# baseline (speedup 1.0000x reference)
import functools

import jax
import jax.numpy as jnp
from jax import lax
from jax.experimental import pallas as pl
from jax.experimental.pallas import tpu as pltpu
from jax.experimental.pallas import tpu_sc as plsc

VOCAB = 100000
D = 1024
BATCH = 4
SEQ = 8192
TOT = BATCH * SEQ

_info = plsc.get_sparse_core_info()
NC = _info.num_cores
NS = _info.num_subcores
NW = NC * NS
BPW = TOT // NW
WPR = SEQ // BPW

CH0 = 64
CH1 = 56
_SIZES = [CH0 if i % 2 == 0 else CH1 for i in range(17)]
assert sum(_SIZES) == BPW
_STARTS = [sum(_SIZES[:i]) for i in range(17)]
NCHUNK = len(_SIZES)

_mesh = plsc.VectorSubcoreMesh(core_axis_name="c", subcore_axis_name="s")


@functools.partial(
    pl.kernel,
    mesh=_mesh,
    out_type=jax.ShapeDtypeStruct((TOT, D), jnp.float32),
    scratch_types=[
        pltpu.VMEM((BPW,), jnp.int32),
        pltpu.VMEM((CH0, D), jnp.float32),
        pltpu.VMEM((CH1, D), jnp.float32),
        pltpu.SemaphoreType.DMA,
        pltpu.SemaphoreType.DMA,
        pltpu.SemaphoreType.DMA,
        pltpu.SemaphoreType.DMA,
    ],
)
def _sc_embed(idx_hbm, table_hbm, out_hbm, idx_v, buf0, buf1, g0, g1, s0, s1):
    wid = lax.axis_index("s") * NC + lax.axis_index("c")
    base = wid * BPW
    bufs = (buf0, buf1)
    gsems = (g0, g1)
    ssems = (s0, s1)

    def gather(c):
        b = c % 2
        return pltpu.make_async_copy(
            table_hbm.at[idx_v.at[pl.ds(_STARTS[c], _SIZES[c])]],
            bufs[b], gsems[b])

    def store(c):
        b = c % 2
        return pltpu.make_async_copy(
            bufs[b], out_hbm.at[pl.ds(base + _STARTS[c], _SIZES[c])],
            ssems[b])

    pltpu.sync_copy(idx_hbm.at[wid // WPR, pl.ds((wid % WPR) * BPW, BPW)],
                    idx_v)

    gather(0).start()
    gather(1).start()
    for c in range(NCHUNK):
        gather(c).wait()
        store(c).start()
        if c + 2 < NCHUNK:
            store(c).wait()
            gather(c + 2).start()
    store(NCHUNK - 2).wait()
    store(NCHUNK - 1).wait()


def kernel(input_ids, table):
    out = _sc_embed(input_ids.astype(jnp.int32), table)
    return out.reshape(BATCH, SEQ, D)


# speedup vs baseline: 1.0182x; 1.0182x over previous
import functools

import jax
import jax.numpy as jnp
from jax import lax
from jax.experimental import pallas as pl
from jax.experimental.pallas import tpu as pltpu
from jax.experimental.pallas import tpu_sc as plsc

VOCAB = 100000
D = 1024
BATCH = 4
SEQ = 8192
TOT = BATCH * SEQ

_info = plsc.get_sparse_core_info()
NC = _info.num_cores
NS = _info.num_subcores
NW = NC * NS
BPW = TOT // NW
WPR = SEQ // BPW
CH = 32
NCHUNK = BPW // CH
NBUF = 3
LOOKAHEAD = 1

_mesh = plsc.VectorSubcoreMesh(core_axis_name="c", subcore_axis_name="s")


@functools.partial(
    pl.kernel,
    mesh=_mesh,
    out_type=jax.ShapeDtypeStruct((TOT, D), jnp.float32),
    scratch_types=[
        pltpu.VMEM((BPW,), jnp.int32),
        pltpu.VMEM((NBUF, CH, D), jnp.float32),
        pltpu.SemaphoreType.DMA,
        pltpu.SemaphoreType.DMA,
        pltpu.SemaphoreType.DMA,
        pltpu.SemaphoreType.DMA,
        pltpu.SemaphoreType.DMA,
        pltpu.SemaphoreType.DMA,
    ],
)
def _sc_embed(idx_hbm, table_hbm, out_hbm, idx_v, bufs,
              g0, g1, g2, s0, s1, s2):
    wid = lax.axis_index("s") * NC + lax.axis_index("c")
    base = wid * BPW
    gsems = (g0, g1, g2)
    ssems = (s0, s1, s2)

    def gather(c, b):
        return pltpu.make_async_copy(
            table_hbm.at[idx_v.at[pl.ds(c * CH, CH)]], bufs.at[b], gsems[b])

    def store(c, b):
        return pltpu.make_async_copy(
            bufs.at[b], out_hbm.at[pl.ds(base + c * CH, CH)], ssems[b])

    pltpu.sync_copy(idx_hbm.at[wid // WPR, pl.ds((wid % WPR) * BPW, BPW)],
                    idx_v)

    for c in range(LOOKAHEAD):
        gather(c, c % NBUF).start()

    def chunk_body(c, _):
        cn = c + LOOKAHEAD

        @pl.when(cn < NCHUNK)
        def _issue_next():
            for b in range(NBUF):

                @pl.when((cn % NBUF) == b)
                def _g():
                    @pl.when(cn >= NBUF)
                    def _free_buf():
                        store(cn - NBUF, b).wait()

                    gather(cn, b).start()

        for b in range(NBUF):

            @pl.when((c % NBUF) == b)
            def _cur():
                gather(c, b).wait()
                store(c, b).start()

        return _

    lax.fori_loop(0, NCHUNK, chunk_body, None)

    for c in range(NCHUNK - NBUF, NCHUNK):
        store(c, c % NBUF).wait()


def kernel(input_ids, table):
    out = _sc_embed(input_ids.astype(jnp.int32), table)
    return out.reshape(BATCH, SEQ, D)


# speedup vs baseline: 1.0206x; 1.0023x over previous
import functools

import jax
import jax.numpy as jnp
from jax import lax
from jax.experimental import pallas as pl
from jax.experimental.pallas import tpu as pltpu
from jax.experimental.pallas import tpu_sc as plsc

VOCAB = 100000
D = 1024
BATCH = 4
SEQ = 8192
TOT = BATCH * SEQ

_info = plsc.get_sparse_core_info()
NC = _info.num_cores
NS = _info.num_subcores
NW = NC * NS
BPW = TOT // NW
WPR = SEQ // BPW
CH = 32
NCHUNK = BPW // CH
NBUF = 3
LOOKAHEAD = 1

_mesh = plsc.VectorSubcoreMesh(core_axis_name="c", subcore_axis_name="s")


@functools.partial(
    pl.kernel,
    mesh=_mesh,
    out_type=jax.ShapeDtypeStruct((TOT, D), jnp.float32),
    scratch_types=[
        pltpu.VMEM((BPW,), jnp.int32),
        pltpu.VMEM((NBUF, CH, D), jnp.float32),
        pltpu.SemaphoreType.DMA,
        pltpu.SemaphoreType.DMA,
        pltpu.SemaphoreType.DMA,
        pltpu.SemaphoreType.DMA,
        pltpu.SemaphoreType.DMA,
        pltpu.SemaphoreType.DMA,
    ],
)
def _sc_embed(idx_hbm, table_hbm, out_hbm, idx_v, bufs,
              g0, g1, g2, s0, s1, s2):
    wid = lax.axis_index("s") * NC + lax.axis_index("c")
    base = wid * BPW
    gsems = (g0, g1, g2)
    ssems = (s0, s1, s2)

    def gather(c, b):
        cc = lax.rem(c + wid, NCHUNK)
        return pltpu.make_async_copy(
            table_hbm.at[idx_v.at[pl.ds(cc * CH, CH)]], bufs.at[b], gsems[b])

    def store(c, b):
        cc = lax.rem(c + wid, NCHUNK)
        return pltpu.make_async_copy(
            bufs.at[b], out_hbm.at[pl.ds(base + cc * CH, CH)], ssems[b])

    pltpu.sync_copy(idx_hbm.at[wid // WPR, pl.ds((wid % WPR) * BPW, BPW)],
                    idx_v)

    for c in range(LOOKAHEAD):
        gather(c, c % NBUF).start()

    def chunk_body(c, _):
        cn = c + LOOKAHEAD

        @pl.when(cn < NCHUNK)
        def _issue_next():
            for b in range(NBUF):

                @pl.when((cn % NBUF) == b)
                def _g():
                    @pl.when(cn >= NBUF)
                    def _free_buf():
                        store(cn - NBUF, b).wait()

                    gather(cn, b).start()

        for b in range(NBUF):

            @pl.when((c % NBUF) == b)
            def _cur():
                gather(c, b).wait()
                store(c, b).start()

        return _

    lax.fori_loop(0, NCHUNK, chunk_body, None)

    for c in range(NCHUNK - NBUF, NCHUNK):
        store(c, c % NBUF).wait()


def kernel(input_ids, table):
    out = _sc_embed(input_ids.astype(jnp.int32), table)
    return out.reshape(BATCH, SEQ, D)
